# baseline (speedup 1.0000x reference)
import functools

import jax
import jax.numpy as jnp
from jax import lax
from jax.experimental import pallas as pl
from jax.experimental.pallas import tpu as pltpu
from jax.experimental.pallas import tpu_sc as plsc

N = 10000
E = 320000
D = 128
NC = 2
NS = 16
CL = 128
CH = (E // NS + CL - 1) // CL
EPT = CH * CL
EP = NS * EPT
NP = 10240
RPT = NP // NS

_mesh = plsc.VectorSubcoreMesh(core_axis_name="c", subcore_axis_name="s")


@functools.partial(
    pl.kernel,
    mesh=_mesh,
    out_type=jax.ShapeDtypeStruct((NC, 2, NP), jnp.float32),
    scratch_types=[
        pltpu.VMEM((CH, CL), jnp.int32),
        pltpu.VMEM((CL,), jnp.float32),
        pltpu.VMEM((RPT,), jnp.float32),
        pltpu.VMEM_SHARED((NP,), jnp.float32),
        pltpu.VMEM_SHARED((NP,), jnp.float32),
    ],
)
def _sc_degrees(degsrc_hbm, dst_hbm, out_hbm, idx_v, ones_v, stage_v,
                degs_sh, degd_sh):
    c = lax.axis_index("c")
    s = lax.axis_index("s")

    def _fill(i, _):
        stage_v[pl.ds(i * 16, 16)] = jnp.zeros((16,), jnp.float32)
        return 0
    lax.fori_loop(0, RPT // 16, _fill, 0)
    def _fill1(i, _):
        ones_v[pl.ds(i * 16, 16)] = jnp.ones((16,), jnp.float32)
        return 0
    lax.fori_loop(0, CL // 16, _fill1, 0)

    pltpu.sync_copy(stage_v, degs_sh.at[pl.ds(s * RPT, RPT)])
    pltpu.sync_copy(stage_v, degd_sh.at[pl.ds(s * RPT, RPT)])
    plsc.subcore_barrier()

    pltpu.sync_copy(degsrc_hbm.at[c, s], idx_v)
    def _cnt_s(j, _):
        pltpu.sync_copy(ones_v, degs_sh.at[idx_v.at[j]], add=True)
        return 0
    lax.fori_loop(0, CH, _cnt_s, 0)

    pltpu.sync_copy(dst_hbm.at[c, s], idx_v)
    def _cnt_d(j, _):
        pltpu.sync_copy(ones_v, degd_sh.at[idx_v.at[j]], add=True)
        return 0
    lax.fori_loop(0, CH, _cnt_d, 0)

    plsc.subcore_barrier()

    pltpu.sync_copy(degs_sh.at[pl.ds(s * RPT, RPT)], stage_v)
    pltpu.sync_copy(stage_v, out_hbm.at[c, 0, pl.ds(s * RPT, RPT)])
    pltpu.sync_copy(degd_sh.at[pl.ds(s * RPT, RPT)], stage_v)
    pltpu.sync_copy(stage_v, out_hbm.at[c, 1, pl.ds(s * RPT, RPT)])


DH = D // 2


@functools.partial(
    pl.kernel,
    mesh=_mesh,
    out_type=jax.ShapeDtypeStruct((2, NC, NP, DH), jnp.float32),
    compiler_params=pltpu.CompilerParams(use_tc_tiling_on_sc=False),
    scratch_types=[
        pltpu.VMEM((CH, CL), jnp.int32),
        pltpu.VMEM((CH, CL), jnp.int32),
        *([pltpu.VMEM((CL, DH), jnp.float32)] * 6),
        pltpu.VMEM_SHARED((NP, DH), jnp.float32),
        *([pltpu.SemaphoreType.DMA] * 12),
    ],
)
def _sc_aggregate(h_hbm, src_hbm, dst_hbm, out_hbm, src_v, dst_v, *rest):
    bufs = rest[:6]
    agg_sh = rest[6]
    gss = rest[7:13]
    sss = rest[13:19]
    buf0 = bufs[0]
    c = lax.axis_index("c")
    s = lax.axis_index("s")

    pltpu.sync_copy(dst_hbm.at[c, s], dst_v)

    for p in range(2):
        def _fill(i, _):
            r = i // (DH // 16)
            k = i % (DH // 16)
            buf0[r, pl.ds(k * 16, 16)] = jnp.zeros((16,), jnp.float32)
            return 0
        lax.fori_loop(0, CL * (DH // 16), _fill, 0)
        for b in range(RPT // CL):
            pltpu.sync_copy(buf0, agg_sh.at[pl.ds(s * RPT + b * CL, CL), :])

        pltpu.sync_copy(src_hbm.at[p, c, s], src_v)
        plsc.subcore_barrier()

        UN = 6

        def _step(jj, _):
            b = jj * UN
            gcopies = [
                pltpu.async_copy(h_hbm.at[src_v.at[b + k]], bufs[k], gss[k])
                for k in range(UN)
            ]
            scopies = []
            for k in range(UN):
                gcopies[k].wait()
                scopies.append(
                    pltpu.async_copy(bufs[k], agg_sh.at[dst_v.at[b + k]],
                                     sss[k], add=True))
            for sc in scopies:
                sc.wait()
            return 0

        lax.fori_loop(0, CH // UN, _step, 0)
        tail0 = (CH // UN) * UN
        tgs = [
            pltpu.async_copy(h_hbm.at[src_v.at[j]], bufs[j - tail0],
                             gss[j - tail0])
            for j in range(tail0, CH)
        ]
        tss = []
        for j in range(tail0, CH):
            tgs[j - tail0].wait()
            tss.append(
                pltpu.async_copy(bufs[j - tail0], agg_sh.at[dst_v.at[j]],
                                 sss[j - tail0], add=True))
        for sc in tss:
            sc.wait()
        plsc.subcore_barrier()

        for b in range(RPT // CL):
            r0 = s * RPT + b * CL
            pltpu.sync_copy(agg_sh.at[pl.ds(r0, CL), :], buf0)
            pltpu.sync_copy(buf0, out_hbm.at[p, c, pl.ds(r0, CL), :])
        plsc.subcore_barrier()


def _tc_scale_body(deg_ref, x_ref, h_ref, rin_ref):
    g = pl.program_id(0)
    i = pl.program_id(1)
    blk = x_ref.shape[0]
    od = deg_ref[g, 0, pl.ds(i * blk, blk)]
    scale = lax.rsqrt(jnp.maximum(od, 1.0))
    h = x_ref[...] * scale[:, None]
    h_ref[0] = h[:, :D // 2]
    h_ref[1] = h[:, D // 2:]
    ind = deg_ref[g, 1, :]
    rin_ref[0, 0] = lax.rsqrt(jnp.maximum(ind, 1.0))


def _tc_scale(deg, x, blk=512):
    nb = NP // blk
    return pl.pallas_call(
        _tc_scale_body,
        grid=(NC, nb),
        in_specs=[
            pl.BlockSpec((NC, 2, NP), lambda g, i: (0, 0, 0)),
            pl.BlockSpec((blk, D), lambda g, i: (i, 0)),
        ],
        out_specs=[
            pl.BlockSpec((2, blk, D // 2), lambda g, i: (0, g * nb + i, 0)),
            pl.BlockSpec((1, 1, NP), lambda g, i: (g, 0, 0)),
        ],
        out_shape=[
            jax.ShapeDtypeStruct((2, NC * NP, D // 2), jnp.float32),
            jax.ShapeDtypeStruct((NC, 1, NP), jnp.float32),
        ],
    )(deg, x)


def _tc_feat_body(agg_ref, rin_ref, w_ref, b_ref, fcw_ref, fcb_ref,
                  out_ref, sp_ref, acc_ref):
    i = pl.program_id(0)
    blk = agg_ref.shape[2]

    @pl.when(i == 0)
    def _():
        acc_ref[...] = jnp.zeros_like(acc_ref)

    row0 = i * blk
    rows = row0 + lax.broadcasted_iota(jnp.int32, (blk, 1), 0)
    mask = rows < N

    for g in range(NC):
        scale = rin_ref[g, 0, pl.ds(row0, blk)][:, None]
        a = jnp.concatenate([agg_ref[0, g], agg_ref[1, g]], axis=-1) * scale
        out = jnp.dot(a, w_ref[g], preferred_element_type=jnp.float32)
        out = out + b_ref[g]
        out_ref[g] = out
        u = jnp.tanh(
            jax.lax.dot_general(out, fcw_ref[0],
                                (((1,), (1,)), ((), ())),
                                preferred_element_type=jnp.float32)
            + fcb_ref[0])
        u = jnp.where(mask, u, 0.0)
        acc_ref[g, :] += jnp.sum(u, axis=0)

    sp_ref[...] = acc_ref[...] * (1.0 / N)


def _tc_feat(agg, rin, w, b, fcw, fcb, blk=512):
    nb = NP // blk
    return pl.pallas_call(
        _tc_feat_body,
        grid=(nb,),
        in_specs=[
            pl.BlockSpec((2, NC, blk, DH), lambda i: (0, 0, i, 0)),
            pl.BlockSpec((NC, 1, NP), lambda i: (0, 0, 0)),
            pl.BlockSpec((NC, D, D), lambda i: (0, 0, 0)),
            pl.BlockSpec((NC, 1, D), lambda i: (0, 0, 0)),
            pl.BlockSpec((1, D, D), lambda i: (0, 0, 0)),
            pl.BlockSpec((1, D), lambda i: (0, 0)),
        ],
        out_specs=[
            pl.BlockSpec((NC, blk, D), lambda i: (0, i, 0)),
            pl.BlockSpec((NC, D), lambda i: (0, 0)),
        ],
        out_shape=[
            jax.ShapeDtypeStruct((NC, NP, D), jnp.float32),
            jax.ShapeDtypeStruct((NC, D), jnp.float32),
        ],
        scratch_shapes=[pltpu.VMEM((NC, D), jnp.float32)],
    )(agg, rin, w, b, fcw, fcb)


def _tc_combine_body(outs_ref, sp_ref, att_ref, hb_ref, res_ref):
    l0 = jnp.sum(att_ref[0, 0, :] * sp_ref[0, 0, :])
    l1 = jnp.sum(att_ref[0, 1, :] * sp_ref[0, 1, :])
    m = jnp.maximum(l0, l1)
    e0 = jnp.exp(l0 - m)
    e1 = jnp.exp(l1 - m)
    inv = 1.0 / (e0 + e1)
    res_ref[...] = (outs_ref[0] * (e0 * inv) + outs_ref[1] * (e1 * inv)
                    + hb_ref[0])


def _tc_combine(outs, sp, att, hb, blk=512):
    nb = NP // blk
    return pl.pallas_call(
        _tc_combine_body,
        grid=(nb,),
        in_specs=[
            pl.BlockSpec((NC, blk, D), lambda i: (0, i, 0)),
            pl.BlockSpec((1, NC, D), lambda i: (0, 0, 0)),
            pl.BlockSpec((1, NC, D), lambda i: (0, 0, 0)),
            pl.BlockSpec((1, D), lambda i: (0, 0)),
        ],
        out_specs=pl.BlockSpec((blk, D), lambda i: (i, 0)),
        out_shape=jax.ShapeDtypeStruct((N, D), jnp.float32),
    )(outs, sp, att, hb)


def _prep_idx(idx, pad_value, offset):
    padn = EP - E
    p = jnp.concatenate(
        [idx + offset, jnp.full((padn,), pad_value, jnp.int32)])
    return p.reshape(NS, CH, CL)


def kernel(inputs, edge_index0, edge_index1, W0, b0, W1, b1, att0, att1,
           fc_W, fc_b, h_bias):
    src0, dst0 = edge_index0[0], edge_index0[1]
    src1, dst1 = edge_index1[0], edge_index1[1]

    degsrc = jnp.stack([_prep_idx(src0, N + 8, 0), _prep_idx(src1, N + 8, 0)])
    dst = jnp.stack([_prep_idx(dst0, N, 0), _prep_idx(dst1, N, 0)])
    srcg = jnp.stack([_prep_idx(src0, 0, 0), _prep_idx(src1, 0, NP)])

    deg = _sc_degrees(degsrc, dst)

    h2, rin = _tc_scale(deg, inputs)

    srcg2 = jnp.stack([srcg, srcg + NC * NP])
    aggh = _sc_aggregate(h2.reshape(2 * NC * NP, DH), srcg2, dst)

    w = jnp.stack([W0, W1])
    b = jnp.stack([b0, b1])[:, None, :]
    outs, sp = _tc_feat(aggh, rin, w, b,
                        fc_W[None], fc_b[None])

    att = jnp.stack([att0[0], att1[0]])[None]
    res = _tc_combine(outs, sp[None], att, h_bias[None])
    return res


# speedup vs baseline: 1.0018x; 1.0018x over previous
import functools

import jax
import jax.numpy as jnp
from jax import lax
from jax.experimental import pallas as pl
from jax.experimental.pallas import tpu as pltpu
from jax.experimental.pallas import tpu_sc as plsc

N = 10000
E = 320000
D = 128
NC = 2
NS = 16
CL = 128
CH = (E // NS + CL - 1) // CL
EPT = CH * CL
EP = NS * EPT
NP = 10240
RPT = NP // NS

_mesh = plsc.VectorSubcoreMesh(core_axis_name="c", subcore_axis_name="s")


DH = D // 2
UN = 5


def _fast_rsqrt(v):
    i = plsc.bitcast(v, jnp.int32)
    i = jnp.int32(0x5F3759DF) - lax.shift_right_arithmetic(i, 1)
    y = plsc.bitcast(i, jnp.float32)
    for _ in range(3):
        y = y * (1.5 - 0.5 * v * y * y)
    return y


@functools.partial(
    pl.kernel,
    mesh=_mesh,
    out_type=(
        jax.ShapeDtypeStruct((2 * NC * NP, DH), jnp.float32),
        jax.ShapeDtypeStruct((NC, NP), jnp.float32),
        jax.ShapeDtypeStruct((2, NC, NP, DH), jnp.float32),
    ),
    compiler_params=pltpu.CompilerParams(use_tc_tiling_on_sc=False,
                                         needs_layout_passes=False),
    scratch_types=[
        pltpu.VMEM((CH, CL), jnp.int32),
        pltpu.VMEM((CH, CL), jnp.int32),
        *([pltpu.VMEM((CL, DH), jnp.float32)] * UN),
        pltpu.VMEM((RPT,), jnp.float32),
        pltpu.VMEM((CL,), jnp.float32),
        pltpu.VMEM_SHARED((NP, DH), jnp.float32),
        pltpu.VMEM_SHARED((NP,), jnp.float32),
        pltpu.VMEM_SHARED((NP,), jnp.float32),
        *([pltpu.SemaphoreType.DMA] * (2 * UN)),
    ],
)
def _sc_fused(x_hbm, degsrc_hbm, dst_hbm, srcg_hbm,
              h_hbm, indeg_hbm, agg_hbm, *rest):
    src_v = rest[0]
    dst_v = rest[1]
    bufs = rest[2:2 + UN]
    odr_v = rest[2 + UN]
    ones_v = rest[3 + UN]
    agg_sh = rest[4 + UN]
    degs_sh = rest[5 + UN]
    degd_sh = rest[6 + UN]
    gss = rest[7 + UN:7 + 2 * UN]
    sss = rest[7 + 2 * UN:7 + 3 * UN]
    c = lax.axis_index("c")
    s = lax.axis_index("s")

    def _fill0(i, _):
        odr_v[pl.ds(i * 16, 16)] = jnp.zeros((16,), jnp.float32)
        return 0
    lax.fori_loop(0, RPT // 16, _fill0, 0)

    def _fill1(i, _):
        ones_v[pl.ds(i * 16, 16)] = jnp.ones((16,), jnp.float32)
        return 0
    lax.fori_loop(0, CL // 16, _fill1, 0)

    pltpu.sync_copy(odr_v, degs_sh.at[pl.ds(s * RPT, RPT)])
    pltpu.sync_copy(odr_v, degd_sh.at[pl.ds(s * RPT, RPT)])
    plsc.subcore_barrier()

    pltpu.sync_copy(degsrc_hbm.at[c, s], src_v)

    def _cnt_s(j, _):
        pltpu.sync_copy(ones_v, degs_sh.at[src_v.at[j]], add=True)
        return 0
    lax.fori_loop(0, CH, _cnt_s, 0)

    pltpu.sync_copy(dst_hbm.at[c, s], dst_v)

    def _cnt_d(j, _):
        pltpu.sync_copy(ones_v, degd_sh.at[dst_v.at[j]], add=True)
        return 0
    lax.fori_loop(0, CH, _cnt_d, 0)

    plsc.subcore_barrier()

    pltpu.sync_copy(degd_sh.at[pl.ds(s * RPT, RPT)], odr_v)
    pltpu.sync_copy(odr_v, indeg_hbm.at[c, pl.ds(s * RPT, RPT)])

    pltpu.sync_copy(degs_sh.at[pl.ds(s * RPT, RPT)], odr_v)

    def _rsq(k, _):
        v = jnp.maximum(odr_v[pl.ds(k * 16, 16)], 1.0)
        odr_v[pl.ds(k * 16, 16)] = _fast_rsqrt(v)
        return 0
    lax.fori_loop(0, RPT // 16, _rsq, 0)

    for p in range(2):
        hbase = (p * NC + c) * NP + s * RPT
        for b in range(RPT // CL):
            pltpu.sync_copy(
                x_hbm.at[pl.ds(s * RPT + b * CL, CL), pl.ds(p * DH, DH)],
                bufs[0])

            def _scale_grp(g16, _):
                ov = odr_v[pl.ds(b * CL + g16 * 16, 16)]
                for r in range(16):
                    sc_ = ov[r]
                    row = g16 * 16 + r
                    for k in range(DH // 16):
                        bufs[0][row, pl.ds(k * 16, 16)] = (
                            bufs[0][row, pl.ds(k * 16, 16)] * sc_)
                return 0
            lax.fori_loop(0, CL // 16, _scale_grp, 0)
            pltpu.sync_copy(bufs[0], h_hbm.at[pl.ds(hbase + b * CL, CL), :])

        def _fillb(i, _):
            r = i // (DH // 16)
            k = i % (DH // 16)
            bufs[1][r, pl.ds(k * 16, 16)] = jnp.zeros((16,), jnp.float32)
            return 0
        lax.fori_loop(0, CL * (DH // 16), _fillb, 0)
        for b in range(RPT // CL):
            pltpu.sync_copy(bufs[1], agg_sh.at[pl.ds(s * RPT + b * CL, CL), :])

        pltpu.sync_copy(srcg_hbm.at[p, c, s], src_v)
        plsc.subcore_barrier()

        def _step(jj, _):
            b = jj * UN
            gcopies = [
                pltpu.async_copy(h_hbm.at[src_v.at[b + k]], bufs[k], gss[k])
                for k in range(UN)
            ]
            scopies = []
            for k in range(UN):
                gcopies[k].wait()
                scopies.append(
                    pltpu.async_copy(bufs[k], agg_sh.at[dst_v.at[b + k]],
                                     sss[k], add=True))
            for sc in scopies:
                sc.wait()
            return 0

        lax.fori_loop(0, CH // UN, _step, 0)
        tail0 = (CH // UN) * UN
        tgs = [
            pltpu.async_copy(h_hbm.at[src_v.at[j]], bufs[j - tail0],
                             gss[j - tail0])
            for j in range(tail0, CH)
        ]
        tss = []
        for j in range(tail0, CH):
            tgs[j - tail0].wait()
            tss.append(
                pltpu.async_copy(bufs[j - tail0], agg_sh.at[dst_v.at[j]],
                                 sss[j - tail0], add=True))
        for sc in tss:
            sc.wait()
        plsc.subcore_barrier()

        for b in range(RPT // CL):
            r0 = s * RPT + b * CL
            pltpu.sync_copy(agg_sh.at[pl.ds(r0, CL), :], bufs[0])
            pltpu.sync_copy(bufs[0], agg_hbm.at[p, c, pl.ds(r0, CL), :])
        plsc.subcore_barrier()


def _tc_feat_body(agg_ref, rin_ref, w_ref, b_ref, fcw_ref, fcb_ref,
                  out_ref, sp_ref, acc_ref):
    i = pl.program_id(0)
    blk = agg_ref.shape[2]

    @pl.when(i == 0)
    def _():
        acc_ref[...] = jnp.zeros_like(acc_ref)

    row0 = i * blk
    rows = row0 + lax.broadcasted_iota(jnp.int32, (blk, 1), 0)
    mask = rows < N

    for g in range(NC):
        scale = lax.rsqrt(jnp.maximum(
            rin_ref[g, 0, pl.ds(row0, blk)], 1.0))[:, None]
        a = jnp.concatenate([agg_ref[0, g], agg_ref[1, g]], axis=-1) * scale
        out = jnp.dot(a, w_ref[g], preferred_element_type=jnp.float32)
        out = out + b_ref[g]
        out_ref[g] = out
        u = jnp.tanh(
            jax.lax.dot_general(out, fcw_ref[0],
                                (((1,), (1,)), ((), ())),
                                preferred_element_type=jnp.float32)
            + fcb_ref[0])
        u = jnp.where(mask, u, 0.0)
        acc_ref[g, :] += jnp.sum(u, axis=0)

    sp_ref[...] = acc_ref[...] * (1.0 / N)


def _tc_feat(agg, rin, w, b, fcw, fcb, blk=512):
    nb = NP // blk
    return pl.pallas_call(
        _tc_feat_body,
        grid=(nb,),
        in_specs=[
            pl.BlockSpec((2, NC, blk, DH), lambda i: (0, 0, i, 0)),
            pl.BlockSpec((NC, 1, NP), lambda i: (0, 0, 0)),
            pl.BlockSpec((NC, D, D), lambda i: (0, 0, 0)),
            pl.BlockSpec((NC, 1, D), lambda i: (0, 0, 0)),
            pl.BlockSpec((1, D, D), lambda i: (0, 0, 0)),
            pl.BlockSpec((1, D), lambda i: (0, 0)),
        ],
        out_specs=[
            pl.BlockSpec((NC, blk, D), lambda i: (0, i, 0)),
            pl.BlockSpec((NC, D), lambda i: (0, 0)),
        ],
        out_shape=[
            jax.ShapeDtypeStruct((NC, NP, D), jnp.float32),
            jax.ShapeDtypeStruct((NC, D), jnp.float32),
        ],
        scratch_shapes=[pltpu.VMEM((NC, D), jnp.float32)],
    )(agg, rin, w, b, fcw, fcb)


def _tc_combine_body(outs_ref, sp_ref, att_ref, hb_ref, res_ref):
    l0 = jnp.sum(att_ref[0, 0, :] * sp_ref[0, 0, :])
    l1 = jnp.sum(att_ref[0, 1, :] * sp_ref[0, 1, :])
    m = jnp.maximum(l0, l1)
    e0 = jnp.exp(l0 - m)
    e1 = jnp.exp(l1 - m)
    inv = 1.0 / (e0 + e1)
    res_ref[...] = (outs_ref[0] * (e0 * inv) + outs_ref[1] * (e1 * inv)
                    + hb_ref[0])


def _tc_combine(outs, sp, att, hb, blk=512):
    nb = NP // blk
    return pl.pallas_call(
        _tc_combine_body,
        grid=(nb,),
        in_specs=[
            pl.BlockSpec((NC, blk, D), lambda i: (0, i, 0)),
            pl.BlockSpec((1, NC, D), lambda i: (0, 0, 0)),
            pl.BlockSpec((1, NC, D), lambda i: (0, 0, 0)),
            pl.BlockSpec((1, D), lambda i: (0, 0)),
        ],
        out_specs=pl.BlockSpec((blk, D), lambda i: (i, 0)),
        out_shape=jax.ShapeDtypeStruct((N, D), jnp.float32),
    )(outs, sp, att, hb)


def _prep_idx(idx, pad_value, offset):
    padn = EP - E
    p = jnp.concatenate(
        [idx + offset, jnp.full((padn,), pad_value, jnp.int32)])
    return p.reshape(NS, CH, CL)


def kernel(inputs, edge_index0, edge_index1, W0, b0, W1, b1, att0, att1,
           fc_W, fc_b, h_bias):
    src0, dst0 = edge_index0[0], edge_index0[1]
    src1, dst1 = edge_index1[0], edge_index1[1]

    degsrc = jnp.stack([_prep_idx(src0, N + 8, 0), _prep_idx(src1, N + 8, 0)])
    dst = jnp.stack([_prep_idx(dst0, N, 0), _prep_idx(dst1, N, 0)])
    srcg = jnp.stack([_prep_idx(src0, 0, 0), _prep_idx(src1, 0, NP)])

    srcg2 = jnp.stack([srcg, srcg + NC * NP])

    xpad = jnp.pad(inputs, ((0, NP - N), (0, 0)))
    _h, indeg, aggh = _sc_fused(xpad, degsrc, dst, srcg2)

    w = jnp.stack([W0, W1])
    b = jnp.stack([b0, b1])[:, None, :]
    outs, sp = _tc_feat(aggh, indeg[:, None, :], w, b,
                        fc_W[None], fc_b[None])

    att = jnp.stack([att0[0], att1[0]])[None]
    res = _tc_combine(outs, sp[None], att, h_bias[None])
    return res
